# baseline (device time: 13846 ns/iter reference)
import jax
import jax.numpy as jnp
from jax import lax
from jax.experimental import pallas as pl
from jax.experimental.pallas import tpu as pltpu


def kernel(partial, resid, gamma):
    _, m, d = partial.shape

    def body(p_ref, r_ref, g_ref, out_ref, send_buf, recv_buf, send_sem, recv_sem):
        my_x = lax.axis_index("x")
        my_y = lax.axis_index("y")
        nbr = (1 - my_x, my_y)

        barrier = pltpu.get_barrier_semaphore()
        pl.semaphore_signal(
            barrier, inc=1, device_id=nbr, device_id_type=pl.DeviceIdType.MESH
        )
        pl.semaphore_wait(barrier, 1)

        send_buf[...] = p_ref[0].astype(jnp.bfloat16)
        rdma = pltpu.make_async_remote_copy(
            src_ref=send_buf,
            dst_ref=recv_buf,
            send_sem=send_sem,
            recv_sem=recv_sem,
            device_id=nbr,
            device_id_type=pl.DeviceIdType.MESH,
        )
        rdma.start()
        rdma.wait()

        y = p_ref[0].astype(jnp.float32) + recv_buf[...].astype(jnp.float32) + r_ref[...]
        rms = jnp.sqrt(jnp.mean(y * y, axis=-1, keepdims=True) + 1e-6)
        out_ref[...] = y / rms * g_ref[...]

    return pl.pallas_call(
        body,
        out_shape=jax.ShapeDtypeStruct((m, d), jnp.float32),
        in_specs=[
            pl.BlockSpec(memory_space=pltpu.VMEM),
            pl.BlockSpec(memory_space=pltpu.VMEM),
            pl.BlockSpec(memory_space=pltpu.VMEM),
        ],
        out_specs=pl.BlockSpec(memory_space=pltpu.VMEM),
        scratch_shapes=[
            pltpu.VMEM((m, d), jnp.bfloat16),
            pltpu.VMEM((m, d), jnp.bfloat16),
            pltpu.SemaphoreType.DMA,
            pltpu.SemaphoreType.DMA,
        ],
        compiler_params=pltpu.CompilerParams(collective_id=0),
    )(partial, resid, gamma.reshape(1, d))


# device time: 13837 ns/iter; 1.0007x vs baseline; 1.0007x over previous
import jax
import jax.numpy as jnp
from jax import lax
from jax.experimental import pallas as pl
from jax.experimental.pallas import tpu as pltpu

N_CHUNK = 4


def kernel(partial, resid, gamma):
    _, m, d = partial.shape
    half = m // 2
    ch = half // N_CHUNK

    def body(p_ref, r_ref, g_ref, out_ref, send_a, recv_a,
             send_sems_a, recv_sems_a, send_sems_b, recv_sems_b):
        my_x = lax.axis_index("x")
        my_y = lax.axis_index("y")
        x_nbr = (1 - my_x, my_y)
        y_nbr = (my_x, 1 - my_y)
        my_base = my_y * half
        other_base = (1 - my_y) * half

        barrier = pltpu.get_barrier_semaphore()
        for nbr in (x_nbr, y_nbr):
            pl.semaphore_signal(
                barrier, inc=1, device_id=nbr,
                device_id_type=pl.DeviceIdType.MESH,
            )
        pl.semaphore_wait(barrier, 2)

        def a_rdma(c):
            sl = pl.ds(c * ch, ch)
            return pltpu.make_async_remote_copy(
                src_ref=send_a.at[sl],
                dst_ref=recv_a.at[sl],
                send_sem=send_sems_a.at[c],
                recv_sem=recv_sems_a.at[c],
                device_id=x_nbr,
                device_id_type=pl.DeviceIdType.MESH,
            )

        def b_rdma(c, base):
            sl = pl.ds(base + c * ch, ch)
            return pltpu.make_async_remote_copy(
                src_ref=out_ref.at[sl],
                dst_ref=out_ref.at[sl],
                send_sem=send_sems_b.at[c],
                recv_sem=recv_sems_b.at[c],
                device_id=y_nbr,
                device_id_type=pl.DeviceIdType.MESH,
            )

        for c in range(N_CHUNK):
            sl = pl.ds(c * ch, ch)
            send_a[sl] = p_ref[0, pl.ds(my_base + c * ch, ch)].astype(jnp.bfloat16)
            a_rdma(c).start()

        for c in range(N_CHUNK):
            a_rdma(c).wait_recv()
            sl = pl.ds(c * ch, ch)
            rows = pl.ds(my_base + c * ch, ch)
            t = (send_a[sl].astype(jnp.float32)
                 + recv_a[sl].astype(jnp.float32)
                 + r_ref[rows])
            inv = lax.rsqrt(jnp.mean(t * t, axis=-1, keepdims=True) + 1e-6)
            out_ref[rows] = (t * inv * g_ref[...]).astype(jnp.bfloat16)
            b_rdma(c, my_base).start()

        for c in range(N_CHUNK):
            b_rdma(c, other_base).wait_recv()
        for c in range(N_CHUNK):
            a_rdma(c).wait_send()
            b_rdma(c, my_base).wait_send()

    return pl.pallas_call(
        body,
        out_shape=jax.ShapeDtypeStruct((m, d), jnp.bfloat16),
        in_specs=[
            pl.BlockSpec(memory_space=pltpu.VMEM),
            pl.BlockSpec(memory_space=pltpu.VMEM),
            pl.BlockSpec(memory_space=pltpu.VMEM),
        ],
        out_specs=pl.BlockSpec(memory_space=pltpu.VMEM),
        scratch_shapes=[
            pltpu.VMEM((half, d), jnp.bfloat16),
            pltpu.VMEM((half, d), jnp.bfloat16),
            pltpu.SemaphoreType.DMA((N_CHUNK,)),
            pltpu.SemaphoreType.DMA((N_CHUNK,)),
            pltpu.SemaphoreType.DMA((N_CHUNK,)),
            pltpu.SemaphoreType.DMA((N_CHUNK,)),
        ],
        compiler_params=pltpu.CompilerParams(collective_id=0),
    )(partial, resid, gamma.reshape(1, d))
